# baseline (device time: 30692 ns/iter reference)
import jax
import jax.numpy as jnp
from jax import lax
from jax.experimental import pallas as pl
from jax.experimental.pallas import tpu as pltpu

N_Y = 4
T = 512
TPD = T // N_Y
D = 512
E = 8
EPD = E // N_Y
F = 1024

NEG_INF = float("-inf")
_MESH = pl.DeviceIdType.MESH


def _body(x_ref, rt_ref, w1_hbm, w2_hbm, out_ref,
          xg_ref, rg_ref, pay_ref, cbuf_ref, rbuf_ref, w1_ref, w2_ref,
          x_ss, x_rs, rt_ss, rt_rs, py_ss, py_rs, c_ss, c_rs, w_sem):
    my_x = lax.axis_index("x")
    my_y = lax.axis_index("y")
    my_z = lax.axis_index("z")

    w1_dma = pltpu.make_async_copy(w1_hbm, w1_ref, w_sem.at[0])
    w2_dma = pltpu.make_async_copy(w2_hbm, w2_ref, w_sem.at[1])
    w1_dma.start()
    w2_dma.start()
    w1_dma.wait()
    w2_dma.wait()

    barrier = pltpu.get_barrier_semaphore()
    for k in range(N_Y - 1):
        peer = (my_y + 1 + k) % N_Y
        pl.semaphore_signal(barrier, inc=1,
                            device_id=(my_x, peer, my_z),
                            device_id_type=_MESH)
    pl.semaphore_wait(barrier, N_Y - 1)

    rg_ref[my_y] = rt_ref[...]
    xg_ref[my_y] = x_ref[...].astype(jnp.bfloat16)

    sends = []

    def _send(buf_ref, slot, dst_ref, dst_slot, ss, ss_slot, rs, peer):
        r = pltpu.make_async_remote_copy(
            src_ref=buf_ref.at[slot], dst_ref=dst_ref.at[dst_slot],
            send_sem=ss.at[ss_slot], recv_sem=rs.at[slot],
            device_id=(my_x, peer, my_z), device_id_type=_MESH)
        r.start()
        sends.append(r)

    for k in range(N_Y - 1):
        peer = (my_y + 1 + k) % N_Y
        _send(rg_ref, my_y, rg_ref, my_y, rt_ss, k, rt_rs, peer)

    _send(xg_ref, my_y, xg_ref, my_y, x_ss, 0, x_rs, (my_y - 1) % N_Y)

    def _wait_recv(buf_ref, sem_arr, slot):
        pltpu.make_async_remote_copy(
            src_ref=buf_ref.at[slot], dst_ref=buf_ref.at[slot],
            send_sem=sem_arr.at[slot], recv_sem=sem_arr.at[slot],
            device_id=(my_x, my_y, my_z), device_id_type=_MESH,
        ).wait_recv()

    for k in range(N_Y - 1):
        peer = (my_y + 1 + k) % N_Y
        _wait_recv(rg_ref, rt_rs, peer)

    xf = x_ref[...]
    m1 = jnp.full((TPD, 1), NEG_INF, jnp.float32)
    m2 = jnp.full((TPD, 1), NEG_INF, jnp.float32)
    i1 = jnp.zeros((TPD, 1), jnp.int32)
    i2 = jnp.zeros((TPD, 1), jnp.int32)
    for o in range(N_Y):
        g_o = lax.dot_general(
            xf, rg_ref[o], (((1,), (1,)), ((), ())),
            preferred_element_type=jnp.float32,
            precision=lax.Precision.HIGHEST,
        )
        for j in range(EPD):
            e_id = o * EPD + j
            g = g_o[:, j:j + 1]
            beats1 = g > m1
            beats2 = g > m2
            m2 = jnp.where(beats1, m1, jnp.where(beats2, g, m2))
            i2 = jnp.where(beats1, i1, jnp.where(beats2, e_id, i2))
            m1 = jnp.where(beats1, g, m1)
            i1 = jnp.where(beats1, e_id, i1)
    r = jnp.exp(m2 - m1)
    w_top1 = 1.0 / (1.0 + r)
    w_top2 = r / (1.0 + r)

    lanes = lax.broadcasted_iota(jnp.int32, (TPD, E), 1)
    pay = (jnp.where(lanes == 0, i1.astype(jnp.float32), 0.0)
           + jnp.where(lanes == 1, i2.astype(jnp.float32), 0.0)
           + jnp.where(lanes == 2, w_top1, 0.0)
           + jnp.where(lanes == 3, w_top2, 0.0))
    pay_ref[my_y] = pay
    for k in range(N_Y - 1):
        peer = (my_y - 1 - k) % N_Y
        _send(pay_ref, my_y, pay_ref, my_y, py_ss, k, py_rs, peer)
    for k in range(1, N_Y - 1):
        peer = (my_y - 1 - k) % N_Y
        _send(xg_ref, my_y, xg_ref, my_y, x_ss, k, x_rs, peer)

    own_acc = None
    for s in range(N_Y):
        b = (my_y + s) % N_Y
        if s > 0:
            _wait_recv(xg_ref, x_rs, b)
        xb = xg_ref[b]
        ys = []
        for j in range(EPD):
            h1 = jnp.dot(xb.astype(jnp.float32), w1_ref[j],
                         preferred_element_type=jnp.float32)
            h1 = jnp.maximum(h1, 0.0)
            ys.append(jnp.dot(h1, w2_ref[j],
                              preferred_element_type=jnp.float32))
        if s > 0:
            _wait_recv(pay_ref, py_rs, b)
        pay_b = pay_ref[b]
        i1v = pay_b[:, 0:1].astype(jnp.int32)
        i2v = pay_b[:, 1:2].astype(jnp.int32)
        w1v = pay_b[:, 2:3]
        w2v = pay_b[:, 3:4]
        acc = jnp.zeros((TPD, D), jnp.float32)
        for j in range(EPD):
            e_id = my_y * EPD + j
            wt = (w1v * (i1v == e_id).astype(jnp.float32)
                  + w2v * (i2v == e_id).astype(jnp.float32))
            acc = acc + ys[j] * wt
        if s == 0:
            own_acc = acc
        else:
            cbuf_ref[b] = acc.astype(jnp.bfloat16)
            r = pltpu.make_async_remote_copy(
                src_ref=cbuf_ref.at[b], dst_ref=rbuf_ref.at[my_y],
                send_sem=c_ss.at[s - 1], recv_sem=c_rs.at[my_y],
                device_id=(my_x, b, my_z), device_id_type=_MESH)
            r.start()
            sends.append(r)

    total = own_acc
    for k in range(N_Y - 1):
        peer = (my_y - 1 - k) % N_Y
        _wait_recv(rbuf_ref, c_rs, peer)
        total = total + rbuf_ref[peer].astype(jnp.float32)
    out_ref[...] = total

    for snd in sends:
        snd.wait_send()


def kernel(x, router, W1, W2):
    rt = router.T
    return pl.pallas_call(
        _body,
        out_shape=jax.ShapeDtypeStruct((TPD, D), jnp.float32),
        in_specs=[
            pl.BlockSpec(memory_space=pltpu.VMEM),
            pl.BlockSpec(memory_space=pltpu.VMEM),
            pl.BlockSpec(memory_space=pltpu.MemorySpace.HBM),
            pl.BlockSpec(memory_space=pltpu.MemorySpace.HBM),
        ],
        out_specs=pl.BlockSpec(memory_space=pltpu.VMEM),
        scratch_shapes=[
            pltpu.VMEM((N_Y, TPD, D), jnp.bfloat16),
            pltpu.VMEM((N_Y, EPD, D), jnp.float32),
            pltpu.VMEM((N_Y, TPD, E), jnp.float32),
            pltpu.VMEM((N_Y, TPD, D), jnp.bfloat16),
            pltpu.VMEM((N_Y, TPD, D), jnp.bfloat16),
            pltpu.VMEM((EPD, D, F), jnp.float32),
            pltpu.VMEM((EPD, F, D), jnp.float32),
            pltpu.SemaphoreType.DMA((N_Y - 1,)),
            pltpu.SemaphoreType.DMA((N_Y,)),
            pltpu.SemaphoreType.DMA((N_Y - 1,)),
            pltpu.SemaphoreType.DMA((N_Y,)),
            pltpu.SemaphoreType.DMA((N_Y - 1,)),
            pltpu.SemaphoreType.DMA((N_Y,)),
            pltpu.SemaphoreType.DMA((N_Y - 1,)),
            pltpu.SemaphoreType.DMA((N_Y,)),
            pltpu.SemaphoreType.DMA((2,)),
        ],
        compiler_params=pltpu.CompilerParams(collective_id=0),
    )(x, rt, W1, W2)
